# baseline (speedup 1.0000x reference)
import functools

import jax
import jax.numpy as jnp
from jax.experimental import pallas as pl
from jax.experimental.pallas import tpu as pltpu

_EPS = 1e-7
_MIN_NORM = 1e-15
_MAX_NORM = 1e6


def _sinh(t):
    e = jnp.exp(t)
    return 0.5 * (e - 1.0 / e)


def _cosh(t):
    e = jnp.exp(t)
    return 0.5 * (e + 1.0 / e)


def _lane0_mask(ncols):
    return jax.lax.broadcasted_iota(jnp.int32, (1, ncols), 1) == 0


def _rownorm_sq(y):
    return jnp.sum(y * y, axis=-1, keepdims=True)


def _zero_col0(z, m0):
    return jnp.where(m0, 0.0, z)


def _expmap0_proj(u, m0):
    y = _zero_col0(u, m0)
    n = jnp.maximum(jnp.sqrt(_rownorm_sq(y)), _MIN_NORM)
    s = _sinh(jnp.minimum(n, 15.0))
    rest = (s / n) * y
    t = jnp.sqrt(jnp.maximum(1.0 + _rownorm_sq(rest), _EPS))
    return jnp.where(m0, t, rest)


def _logmap0(xh, m0):
    y = _zero_col0(xh, m0)
    n = jnp.maximum(jnp.sqrt(_rownorm_sq(y)), _MIN_NORM)
    th = jnp.maximum(xh[:, 0:1], 1.0 + _EPS)
    arc = jnp.log(th + jnp.sqrt(th * th - 1.0))
    return (arc / n) * y


def _compute_xt(x, w, b, m0):
    h1 = _expmap0_proj(x, m0)
    u = _logmap0(h1, m0)
    mu = jax.lax.dot_general(u, w, (((1,), (1,)), ((), ())),
                             preferred_element_type=jnp.float32)
    res = _expmap0_proj(mu, m0)

    hyp_bias = _expmap0_proj(b, m0)
    ub = _logmap0(hyp_bias, m0)
    x0 = res[:, 0:1]
    y = _zero_col0(res, m0)
    y_norm = jnp.maximum(jnp.sqrt(_rownorm_sq(y)), _MIN_NORM)
    y_unit = y / y_norm
    vv = jnp.where(m0, -y_norm, (1.0 - x0) * y_unit)
    alpha = jnp.sum(y_unit * ub, axis=-1, keepdims=True)
    res2 = ub - alpha * vv
    ux = jnp.sum(y * _zero_col0(res2, m0), axis=-1, keepdims=True)
    b0 = ux / jnp.maximum(x0, _EPS)
    v = jnp.where(m0, b0, res2)
    mdot = _rownorm_sq(v) - 2.0 * v[:, 0:1] * v[:, 0:1]
    normu = jnp.minimum(jnp.sqrt(jnp.maximum(mdot, _EPS)), _MAX_NORM)
    th = jnp.maximum(normu, _MIN_NORM)
    thc = jnp.minimum(th, 15.0)
    result = _cosh(thc) * res + (_sinh(thc) / th) * v
    h2_rest = _zero_col0(result, m0)
    h2_0 = jnp.sqrt(jnp.maximum(1.0 + _rownorm_sq(h2_rest), _EPS))
    h2 = jnp.where(m0, h2_0, result)

    return _logmap0(h2, m0)


def _merged_body(*refs):
    nstream = len(refs) - 5
    adj_refs = refs[:nstream]
    x_ref, w_ref, b_ref, out_ref, xt_ref = refs[nstream:]
    d = x_ref.shape[-1]
    m0 = _lane0_mask(d)

    n = x_ref.shape[0]
    ch = 1000 if n % 1000 == 0 else n

    @pl.when(pl.program_id(0) == 0)
    def _():
        def body(j, carry):
            sl = pl.ds(j * ch, ch)
            xt_ref[sl, :] = _compute_xt(x_ref[sl, :], w_ref[...],
                                        b_ref[...], m0)
            return carry

        jax.lax.fori_loop(0, n // ch, body, 0)

    xt = xt_ref[...]
    bs = adj_refs[0].shape[0]
    for s, adj_ref in enumerate(adj_refs):
        support = jax.lax.dot_general(adj_ref[...], xt,
                                      (((1,), (0,)), ((), ())),
                                      preferred_element_type=jnp.float32)
        h3 = _expmap0_proj(support, m0)
        l = _logmap0(h3, m0)
        lr = jnp.where(l >= 0.0, l, 0.01 * l)
        out_ref[pl.ds(s * bs, bs), :] = _expmap0_proj(lr, m0)


def _pick_block(n, cap):
    for bs in range(cap, 7, -8):
        if n % bs == 0:
            return bs
    return n


@jax.jit
def kernel(x, adj, W, b):
    n, d = x.shape
    b2 = b.reshape(1, d).astype(jnp.float32)

    nstream = 5
    bi = 400
    if n % (nstream * (bi // nstream)) != 0 or (bi // nstream) % 8 != 0:
        nstream, bi = 1, _pick_block(n, 400)
    bs = bi // nstream
    adj_specs = [
        pl.BlockSpec((bs, n), functools.partial(
            lambda s, i: (nstream * i + s, 0), s))
        for s in range(nstream)
    ]
    out = pl.pallas_call(
        _merged_body,
        grid=(n // bi,),
        in_specs=adj_specs + [
            pl.BlockSpec((n, d), lambda i: (0, 0)),
            pl.BlockSpec((d, d), lambda i: (0, 0)),
            pl.BlockSpec((1, d), lambda i: (0, 0)),
        ],
        out_specs=pl.BlockSpec((bi, d), lambda i: (i, 0)),
        out_shape=jax.ShapeDtypeStruct((n, d), jnp.float32),
        scratch_shapes=[pltpu.VMEM((n, d), jnp.float32)],
        compiler_params=pltpu.CompilerParams(
            dimension_semantics=("arbitrary",)),
    )(*([adj] * nstream), x, W, b2)
    return out


# speedup vs baseline: 1.0190x; 1.0190x over previous
import functools

import jax
import jax.numpy as jnp
from jax.experimental import pallas as pl
from jax.experimental.pallas import tpu as pltpu

_EPS = 1e-7
_MIN_NORM = 1e-15
_MAX_NORM = 1e6


def _sinh(t):
    e = jnp.exp(t)
    return 0.5 * (e - 1.0 / e)


def _cosh(t):
    e = jnp.exp(t)
    return 0.5 * (e + 1.0 / e)


def _lane0_mask(ncols):
    return jax.lax.broadcasted_iota(jnp.int32, (1, ncols), 1) == 0


def _rownorm_sq(y):
    return jnp.sum(y * y, axis=-1, keepdims=True)


def _zero_col0(z, m0):
    return jnp.where(m0, 0.0, z)


def _expmap0_proj(u, m0):
    y = _zero_col0(u, m0)
    n = jnp.maximum(jnp.sqrt(_rownorm_sq(y)), _MIN_NORM)
    s = _sinh(jnp.minimum(n, 15.0))
    rest = (s / n) * y
    t = jnp.sqrt(jnp.maximum(1.0 + _rownorm_sq(rest), _EPS))
    return jnp.where(m0, t, rest)


def _logmap0(xh, m0):
    y = _zero_col0(xh, m0)
    n = jnp.maximum(jnp.sqrt(_rownorm_sq(y)), _MIN_NORM)
    th = jnp.maximum(xh[:, 0:1], 1.0 + _EPS)
    arc = jnp.log(th + jnp.sqrt(th * th - 1.0))
    return (arc / n) * y


def _compute_xt(x, w, b, m0):
    h1 = _expmap0_proj(x, m0)
    u = _logmap0(h1, m0)
    mu = jax.lax.dot_general(u, w, (((1,), (1,)), ((), ())),
                             preferred_element_type=jnp.float32)
    res = _expmap0_proj(mu, m0)

    hyp_bias = _expmap0_proj(b, m0)
    ub = _logmap0(hyp_bias, m0)
    x0 = res[:, 0:1]
    y = _zero_col0(res, m0)
    y_norm = jnp.maximum(jnp.sqrt(_rownorm_sq(y)), _MIN_NORM)
    y_unit = y / y_norm
    vv = jnp.where(m0, -y_norm, (1.0 - x0) * y_unit)
    alpha = jnp.sum(y_unit * ub, axis=-1, keepdims=True)
    res2 = ub - alpha * vv
    ux = jnp.sum(y * _zero_col0(res2, m0), axis=-1, keepdims=True)
    b0 = ux / jnp.maximum(x0, _EPS)
    v = jnp.where(m0, b0, res2)
    mdot = _rownorm_sq(v) - 2.0 * v[:, 0:1] * v[:, 0:1]
    normu = jnp.minimum(jnp.sqrt(jnp.maximum(mdot, _EPS)), _MAX_NORM)
    th = jnp.maximum(normu, _MIN_NORM)
    thc = jnp.minimum(th, 15.0)
    result = _cosh(thc) * res + (_sinh(thc) / th) * v
    h2_rest = _zero_col0(result, m0)
    h2_0 = jnp.sqrt(jnp.maximum(1.0 + _rownorm_sq(h2_rest), _EPS))
    h2 = jnp.where(m0, h2_0, result)

    return _logmap0(h2, m0)


def _merged_body(adj_ref, x_ref, w_ref, b_ref, out_ref, xt_ref):
    d = x_ref.shape[-1]
    m0 = _lane0_mask(d)

    n = x_ref.shape[0]
    ch = 1000 if n % 1000 == 0 else n

    @pl.when(pl.program_id(0) == 0)
    def _():
        def body(j, carry):
            sl = pl.ds(j * ch, ch)
            xt_ref[sl, :] = _compute_xt(x_ref[sl, :], w_ref[...],
                                        b_ref[...], m0)
            return carry

        jax.lax.fori_loop(0, n // ch, body, 0)

    support = jax.lax.dot_general(adj_ref[...], xt_ref[...],
                                  (((1,), (0,)), ((), ())),
                                  preferred_element_type=jnp.float32)
    n = jnp.maximum(jnp.sqrt(_rownorm_sq(support)), _MIN_NORM)
    s = _sinh(jnp.minimum(n, 15.0))
    t3 = jnp.sqrt(jnp.maximum(1.0 + s * s, _EPS))
    th3 = jnp.maximum(t3, 1.0 + _EPS)
    arc = jnp.log(th3 + jnp.sqrt(th3 * th3 - 1.0))
    g = arc * (s / jnp.maximum(s, _MIN_NORM)) / n
    l = g * support
    lr = jnp.where(l >= 0.0, l, 0.01 * l)
    n6 = jnp.maximum(jnp.sqrt(_rownorm_sq(lr)), _MIN_NORM)
    s6 = _sinh(jnp.minimum(n6, 15.0))
    rest6 = (s6 / n6) * lr
    t6 = jnp.sqrt(jnp.maximum(1.0 + s6 * s6, _EPS))
    out_ref[...] = jnp.where(m0, t6, rest6)


def _pick_block(n, cap):
    for bs in range(cap, 7, -8):
        if n % bs == 0:
            return bs
    return n


@jax.jit
def kernel(x, adj, W, b):
    n, d = x.shape
    b2 = b.reshape(1, d).astype(jnp.float32)

    bi = _pick_block(n, 400)
    out = pl.pallas_call(
        _merged_body,
        grid=(n // bi,),
        in_specs=[
            pl.BlockSpec((bi, n), lambda i: (i, 0)),
            pl.BlockSpec((n, d), lambda i: (0, 0)),
            pl.BlockSpec((d, d), lambda i: (0, 0)),
            pl.BlockSpec((1, d), lambda i: (0, 0)),
        ],
        out_specs=pl.BlockSpec((bi, d), lambda i: (i, 0)),
        out_shape=jax.ShapeDtypeStruct((n, d), jnp.float32),
        scratch_shapes=[pltpu.VMEM((n, d), jnp.float32)],
        compiler_params=pltpu.CompilerParams(
            dimension_semantics=("arbitrary",)),
    )(adj, x, W, b2)
    return out
